# baseline (speedup 1.0000x reference)
import jax
import jax.numpy as jnp
from jax.experimental import pallas as pl
from jax.experimental.pallas import tpu as pltpu

_BM = 200
_DEPTH = 4


def _support_kernel(feature_ref, w_ref, out_ref, out_bf16_ref):
    acc = jnp.dot(feature_ref[...], w_ref[...], preferred_element_type=jnp.float32)
    sup = jnp.maximum(acc, 0.0)
    out_ref[...] = sup
    out_bf16_ref[...] = sup.astype(jnp.bfloat16)


def _agg_kernel(adj_hbm, supb_ref, sup_ref, b_ref, out_ref, buf, sems):
    n = out_ref.shape[0]
    nchunk = n // _BM

    def copy(c, slot):
        return pltpu.make_async_copy(
            adj_hbm.at[pl.ds(c * _BM, _BM), :], buf.at[slot], sems.at[slot]
        )

    for c in range(_DEPTH):
        copy(c, c).start()

    sup = supb_ref[...]
    bias = b_ref[...]

    def step(c, carry):
        slot = jax.lax.rem(c, _DEPTH)
        copy(c, slot).wait()
        acc = jnp.dot(
            buf[slot].astype(jnp.bfloat16), sup,
            preferred_element_type=jnp.float32,
        )

        @pl.when(c + _DEPTH < nchunk)
        def _():
            copy(c + _DEPTH, slot).start()

        out_ref[pl.ds(c * _BM, _BM), :] = (
            acc + sup_ref[pl.ds(c * _BM, _BM), :] + bias
        )
        return carry

    jax.lax.fori_loop(0, nchunk, step, 0)


def kernel(feature, adj, W, b):
    n, _ = feature.shape
    d = W.shape[1]

    support, support_bf16 = pl.pallas_call(
        _support_kernel,
        out_shape=(
            jax.ShapeDtypeStruct((n, d), jnp.float32),
            jax.ShapeDtypeStruct((n, d), jnp.bfloat16),
        ),
    )(feature, W)

    out = pl.pallas_call(
        _agg_kernel,
        in_specs=[
            pl.BlockSpec(memory_space=pltpu.HBM),
            pl.BlockSpec(memory_space=pltpu.VMEM),
            pl.BlockSpec(memory_space=pltpu.VMEM),
            pl.BlockSpec(memory_space=pltpu.VMEM),
        ],
        out_specs=pl.BlockSpec(memory_space=pltpu.VMEM),
        out_shape=jax.ShapeDtypeStruct((n, d), jnp.float32),
        scratch_shapes=[
            pltpu.VMEM((_DEPTH, _BM, n), jnp.float32),
            pltpu.SemaphoreType.DMA((_DEPTH,)),
        ],
    )(adj, support_bf16, support, b.reshape(1, d))
    return out


# speedup vs baseline: 1.0046x; 1.0046x over previous
import jax
import jax.numpy as jnp
from jax.experimental import pallas as pl
from jax.experimental.pallas import tpu as pltpu

_BM = 200
_DEPTH = 2


def _support_kernel(feature_ref, w_ref, out_ref, out_bf16_ref):
    acc = jnp.dot(feature_ref[...], w_ref[...], preferred_element_type=jnp.float32)
    sup = jnp.maximum(acc, 0.0)
    out_ref[...] = sup
    out_bf16_ref[...] = sup.astype(jnp.bfloat16)


def _agg_kernel(adj_hbm, supb_ref, sup_ref, b_ref, out_ref,
                buf0, buf1, sems0, sems1):
    n = out_ref.shape[0]
    nchunk = n // _BM
    ngroup = nchunk // 2
    bufs = (buf0, buf1)
    sems = (sems0, sems1)

    def copy(s, c, slot):
        return pltpu.make_async_copy(
            adj_hbm.at[pl.ds(c * _BM, _BM), :], bufs[s].at[slot],
            sems[s].at[slot],
        )

    for t in range(_DEPTH):
        for s in range(2):
            copy(s, 2 * t + s, t).start()

    sup = supb_ref[...]
    bias = b_ref[...]

    def step(t, carry):
        slot = jax.lax.rem(t, _DEPTH)
        for s in range(2):
            c = 2 * t + s
            copy(s, c, slot).wait()
            acc = jnp.dot(
                bufs[s][slot].astype(jnp.bfloat16), sup,
                preferred_element_type=jnp.float32,
            )

            @pl.when(t + _DEPTH < ngroup)
            def _(s=s, c=c, slot=slot):
                copy(s, c + 2 * _DEPTH, slot).start()

            out_ref[pl.ds(c * _BM, _BM), :] = (
                acc + sup_ref[pl.ds(c * _BM, _BM), :] + bias
            )
        return carry

    jax.lax.fori_loop(0, ngroup, step, 0)


def kernel(feature, adj, W, b):
    n, _ = feature.shape
    d = W.shape[1]

    support, support_bf16 = pl.pallas_call(
        _support_kernel,
        out_shape=(
            jax.ShapeDtypeStruct((n, d), jnp.float32),
            jax.ShapeDtypeStruct((n, d), jnp.bfloat16),
        ),
    )(feature, W)

    out = pl.pallas_call(
        _agg_kernel,
        in_specs=[
            pl.BlockSpec(memory_space=pltpu.HBM),
            pl.BlockSpec(memory_space=pltpu.VMEM),
            pl.BlockSpec(memory_space=pltpu.VMEM),
            pl.BlockSpec(memory_space=pltpu.VMEM),
        ],
        out_specs=pl.BlockSpec(memory_space=pltpu.VMEM),
        out_shape=jax.ShapeDtypeStruct((n, d), jnp.float32),
        scratch_shapes=[
            pltpu.VMEM((_DEPTH, _BM, n), jnp.float32),
            pltpu.VMEM((_DEPTH, _BM, n), jnp.float32),
            pltpu.SemaphoreType.DMA((_DEPTH,)),
            pltpu.SemaphoreType.DMA((_DEPTH,)),
        ],
    )(adj, support_bf16, support, b.reshape(1, d))
    return out
